# speedup vs baseline: 7.7786x; 7.7786x over previous
import functools

import jax
import jax.numpy as jnp
from jax import lax
from jax.experimental import pallas as pl
from jax.experimental.pallas import tpu as pltpu
from jax.experimental.pallas import tpu_sc as plsc

F = 128
K = 32
NC = 2
NS = 16
NW = NC * NS
NPAD = 10240
NODES_PER_W = NPAD // NW
NB = 4
NUM_BLOCKS = NODES_PER_W // NB
LANES = 16
VPR = F // LANES


def _sc_body(x_hbm, adj_hbm, out_hbm, idx_v, rows_v, acc_v, sem):
    wid = lax.axis_index("s") * NC + lax.axis_index("c")
    node_base = wid * NODES_PER_W

    def block_fn(b, carry):
        base = node_base + b * NB
        pltpu.sync_copy(adj_hbm.at[pl.ds(base * K, NB * K)], idx_v)
        pltpu.async_copy(x_hbm.at[idx_v], rows_v, sem).wait()
        for n in range(NB):
            def k_fn(k, accs, n=n):
                r = n * K + k
                return tuple(accs[j] + rows_v[r, pl.ds(j * LANES, LANES)]
                             for j in range(VPR))
            accs = lax.fori_loop(
                0, K, k_fn,
                tuple(jnp.zeros((LANES,), jnp.float32) for _ in range(VPR)))
            for j in range(VPR):
                acc_v[n, pl.ds(j * LANES, LANES)] = accs[j]
        pltpu.sync_copy(acc_v, out_hbm.at[pl.ds(base, NB)])
        return carry

    lax.fori_loop(0, NUM_BLOCKS, block_fn, 0)


_sc_gather_sum = functools.partial(
    pl.kernel,
    mesh=plsc.VectorSubcoreMesh(core_axis_name="c", subcore_axis_name="s"),
    out_type=jax.ShapeDtypeStruct((NPAD, F), jnp.float32),
    scratch_types=[
        pltpu.VMEM((NB * K,), jnp.int32),
        pltpu.VMEM((NB * K, F), jnp.float32),
        pltpu.VMEM((NB, F), jnp.float32),
        pltpu.SemaphoreType.DMA,
    ],
)(_sc_body)


def _tc_body(x_ref, nei_ref, ws_ref, wn_ref, b_ref, o_ref):
    acc = lax.dot_general(x_ref[...], ws_ref[...], (((1,), (0,)), ((), ())),
                          precision=lax.Precision.HIGHEST,
                          preferred_element_type=jnp.float32)
    acc = acc + lax.dot_general(nei_ref[...], wn_ref[...],
                                (((1,), (0,)), ((), ())),
                                precision=lax.Precision.HIGHEST,
                                preferred_element_type=jnp.float32)
    o_ref[...] = jnp.maximum(acc + b_ref[...], 0.0)


def _tc_fused(x2, nei_sum, W_self, W_nei_scaled, bias, n_rows, bm):
    grid = (n_rows // bm,)
    return pl.pallas_call(
        _tc_body,
        grid=grid,
        in_specs=[
            pl.BlockSpec((bm, F), lambda i: (i, 0)),
            pl.BlockSpec((bm, F), lambda i: (i, 0)),
            pl.BlockSpec((F, F), lambda i: (0, 0)),
            pl.BlockSpec((F, F), lambda i: (0, 0)),
            pl.BlockSpec((1, F), lambda i: (0, 0)),
        ],
        out_specs=pl.BlockSpec((bm, F), lambda i: (i, 0)),
        out_shape=jax.ShapeDtypeStruct((n_rows, F), jnp.float32),
    )(x2, nei_sum, W_self, W_nei_scaled, bias)


def kernel(x, adj, W_self, b_self, W_nei, b_nei):
    B, N, Fd = x.shape
    Kd = adj.shape[-1]
    x2 = x.reshape(N, Fd)
    adj_flat = adj.reshape(-1).astype(jnp.int32)
    adj_pad = jnp.pad(adj_flat, (0, (NPAD - N) * Kd))
    nei_sum = _sc_gather_sum(x2, adj_pad)
    out = _tc_fused(x2, nei_sum, W_self, W_nei * (1.0 / Kd),
                    (b_self + b_nei).reshape(1, Fd), N, 400)
    return out.reshape(B, N, Fd)


# speedup vs baseline: 37.4973x; 4.8206x over previous
import functools

import jax
import jax.numpy as jnp
from jax import lax
from jax.experimental import pallas as pl
from jax.experimental.pallas import tpu as pltpu
from jax.experimental.pallas import tpu_sc as plsc

F = 128
K = 32
NC = 2
NS = 16
NW = NC * NS
NPAD = 10240
NODES_PER_W = NPAD // NW
GROWS = 128
GN = GROWS // K
NSTEPS = NODES_PER_W // GN
RING = 2
NOUTER = NSTEPS // RING
LANES = 16
VPR = F // LANES
N_ROWS = 10240
ROWS_PER_TILE = N_ROWS // NS


def _sc_body(x_hbm, adj_hbm, out_hbm, x_sh, idx_all,
             rows0, rows1, ob0, ob1, sem0, sem1, osem0, osem1):
    rows = (rows0, rows1)
    sems = (sem0, sem1)
    obs = (ob0, ob1)
    osems = (osem0, osem1)
    sid = lax.axis_index("s")
    wid = sid * NC + lax.axis_index("c")
    node_base = wid * NODES_PER_W

    pltpu.sync_copy(x_hbm.at[pl.ds(sid * ROWS_PER_TILE, ROWS_PER_TILE)],
                    x_sh.at[pl.ds(sid * ROWS_PER_TILE, ROWS_PER_TILE)])

    pltpu.sync_copy(adj_hbm.at[wid], idx_all)
    plsc.subcore_barrier()

    def start(g, b):
        pltpu.async_copy(x_sh.at[idx_all.at[g]], rows[b], sems[b])

    for b in range(RING):
        start(b, b)

    def outer(gb, carry):
        for b in range(RING):
            g = gb * RING + b
            pltpu.make_async_copy(
                x_hbm.at[pl.ds(0, GROWS)], rows[b], sems[b]).wait()

            @pl.when(gb > 0)
            def _(b=b):
                pltpu.make_async_copy(
                    obs[b], out_hbm.at[pl.ds(node_base, GN)],
                    osems[b]).wait()

            for n in range(GN):
                def k_fn(k, accs, b=b, n=n):
                    r = n * K + k
                    return tuple(accs[j] + rows[b][r, pl.ds(j * LANES, LANES)]
                                 for j in range(VPR))
                accs = lax.fori_loop(
                    0, K, k_fn,
                    tuple(jnp.zeros((LANES,), jnp.float32)
                          for _ in range(VPR)))
                for j in range(VPR):
                    obs[b][n, pl.ds(j * LANES, LANES)] = accs[j]
            pltpu.async_copy(
                obs[b], out_hbm.at[pl.ds(node_base + g * GN, GN)], osems[b])

            @pl.when(gb < NOUTER - 1)
            def _(g=g, b=b):
                start(g + RING, b)
        return carry

    lax.fori_loop(0, NOUTER, outer, 0)
    for b in range(RING):
        pltpu.make_async_copy(
            obs[b], out_hbm.at[pl.ds(node_base, GN)], osems[b]).wait()


_sc_gather_sum = functools.partial(
    pl.kernel,
    mesh=plsc.VectorSubcoreMesh(core_axis_name="c", subcore_axis_name="s"),
    out_type=jax.ShapeDtypeStruct((NPAD, F), jnp.float32),
    scratch_types=[
        pltpu.VMEM_SHARED((N_ROWS, F), jnp.float32),
        pltpu.VMEM((NSTEPS, GROWS), jnp.int32),
        pltpu.VMEM((GROWS, F), jnp.float32),
        pltpu.VMEM((GROWS, F), jnp.float32),
        pltpu.VMEM((GN, F), jnp.float32),
        pltpu.VMEM((GN, F), jnp.float32),
        pltpu.SemaphoreType.DMA,
        pltpu.SemaphoreType.DMA,
        pltpu.SemaphoreType.DMA,
        pltpu.SemaphoreType.DMA,
    ],
)(_sc_body)


def _tc_body(x_ref, nei_ref, ws_ref, wn_ref, b_ref, o_ref):
    acc = lax.dot_general(x_ref[...], ws_ref[...], (((1,), (0,)), ((), ())),
                          precision=lax.Precision.HIGHEST,
                          preferred_element_type=jnp.float32)
    acc = acc + lax.dot_general(nei_ref[...], wn_ref[...],
                                (((1,), (0,)), ((), ())),
                                precision=lax.Precision.HIGHEST,
                                preferred_element_type=jnp.float32)
    o_ref[...] = jnp.maximum(acc + b_ref[...], 0.0)


def _tc_fused(x2, nei_sum, W_self, W_nei_scaled, bias, n_rows, bm):
    grid = (n_rows // bm,)
    return pl.pallas_call(
        _tc_body,
        grid=grid,
        in_specs=[
            pl.BlockSpec((bm, F), lambda i: (i, 0)),
            pl.BlockSpec((bm, F), lambda i: (i, 0)),
            pl.BlockSpec((F, F), lambda i: (0, 0)),
            pl.BlockSpec((F, F), lambda i: (0, 0)),
            pl.BlockSpec((1, F), lambda i: (0, 0)),
        ],
        out_specs=pl.BlockSpec((bm, F), lambda i: (i, 0)),
        out_shape=jax.ShapeDtypeStruct((n_rows, F), jnp.float32),
    )(x2, nei_sum, W_self, W_nei_scaled, bias)


def kernel(x, adj, W_self, b_self, W_nei, b_nei):
    B, N, Fd = x.shape
    Kd = adj.shape[-1]
    x2 = x.reshape(N, Fd)
    x2p = jnp.pad(x2, ((0, N_ROWS - N), (0, 0)))
    adj_flat = adj.reshape(-1).astype(jnp.int32)
    adj_pad = jnp.pad(adj_flat, (0, (NPAD - N) * Kd)).reshape(NW, NSTEPS, GROWS)
    nei_sum = _sc_gather_sum(x2p, adj_pad)
    out = _tc_fused(x2, nei_sum, W_self, W_nei * (1.0 / Kd),
                    (b_self + b_nei).reshape(1, Fd), N, 400)
    return out.reshape(B, N, Fd)


# speedup vs baseline: 44.8519x; 1.1961x over previous
import functools

import jax
import jax.numpy as jnp
from jax import lax
from jax.experimental import pallas as pl
from jax.experimental.pallas import tpu as pltpu
from jax.experimental.pallas import tpu_sc as plsc

F = 128
K = 32
N_NODES = 10000
NC = 2
NS = 16
NW = NC * NS
NODES_PER_W = 320
GROWS = 128
GN = GROWS // K
NSTEPS = NODES_PER_W // GN
RING = 2
NOUTER = NSTEPS // RING
NOUTER_LAST = (N_NODES - (NW - 1) * NODES_PER_W) // GN // RING
KU = 4
LANES = 16
VPR = F // LANES
ROWS_PER_TILE = 640
LAST_TILE_ROWS = N_NODES - 15 * ROWS_PER_TILE
IDX_PER_W = NODES_PER_W * K
IDX_PER_W_LAST = (N_NODES - (NW - 1) * NODES_PER_W) * K


def _sc_body(x_hbm, adj_hbm, out_hbm, x_sh, idx_all,
             rows0, rows1, ob0, ob1, sem0, sem1, osem0, osem1):
    rows = (rows0, rows1)
    sems = (sem0, sem1)
    obs = (ob0, ob1)
    osems = (osem0, osem1)
    sid = lax.axis_index("s")
    wid = sid * NC + lax.axis_index("c")
    node_base = wid * NODES_PER_W
    n_outer = jnp.where(wid == NW - 1, NOUTER_LAST, NOUTER)

    @pl.when(sid < NS - 1)
    def _():
        pltpu.sync_copy(x_hbm.at[pl.ds(sid * ROWS_PER_TILE, ROWS_PER_TILE)],
                        x_sh.at[pl.ds(sid * ROWS_PER_TILE, ROWS_PER_TILE)])

    @pl.when(sid == NS - 1)
    def _():
        pltpu.sync_copy(
            x_hbm.at[pl.ds((NS - 1) * ROWS_PER_TILE, LAST_TILE_ROWS)],
            x_sh.at[pl.ds((NS - 1) * ROWS_PER_TILE, LAST_TILE_ROWS)])

    @pl.when(wid < NW - 1)
    def _():
        pltpu.sync_copy(adj_hbm.at[pl.ds(wid * IDX_PER_W, IDX_PER_W)],
                        idx_all)

    @pl.when(wid == NW - 1)
    def _():
        pltpu.sync_copy(
            adj_hbm.at[pl.ds((NW - 1) * IDX_PER_W, IDX_PER_W_LAST)],
            idx_all.at[pl.ds(0, IDX_PER_W_LAST)])
    plsc.subcore_barrier()

    def start(g, b):
        pltpu.async_copy(x_sh.at[idx_all.at[pl.ds(g * GROWS, GROWS)]],
                         rows[b], sems[b])

    for b in range(RING):
        start(b, b)

    def outer(gb, carry):
        for b in range(RING):
            g = gb * RING + b
            pltpu.make_async_copy(
                x_hbm.at[pl.ds(0, GROWS)], rows[b], sems[b]).wait()

            @pl.when(gb > 0)
            def _(b=b):
                pltpu.make_async_copy(
                    obs[b], out_hbm.at[pl.ds(node_base, GN)],
                    osems[b]).wait()

            for n in range(GN):
                def k_fn(kq, accs, b=b, n=n):
                    accs = list(accs)
                    for u in range(KU):
                        r = n * K + kq * KU + u
                        for j in range(VPR):
                            accs[j] = accs[j] + rows[b][
                                r, pl.ds(j * LANES, LANES)]
                    return tuple(accs)
                accs = lax.fori_loop(
                    0, K // KU, k_fn,
                    tuple(jnp.zeros((LANES,), jnp.float32)
                          for _ in range(VPR)))
                for j in range(VPR):
                    obs[b][n, pl.ds(j * LANES, LANES)] = accs[j]
            pltpu.async_copy(
                obs[b], out_hbm.at[pl.ds(node_base + g * GN, GN)], osems[b])

            @pl.when(gb < n_outer - 1)
            def _(g=g, b=b):
                start(g + RING, b)
        return carry

    lax.fori_loop(0, n_outer, outer, 0)
    for b in range(RING):
        pltpu.make_async_copy(
            obs[b], out_hbm.at[pl.ds(node_base, GN)], osems[b]).wait()


_sc_gather_sum = functools.partial(
    pl.kernel,
    mesh=plsc.VectorSubcoreMesh(core_axis_name="c", subcore_axis_name="s"),
    out_type=jax.ShapeDtypeStruct((N_NODES, F), jnp.float32),
    scratch_types=[
        pltpu.VMEM_SHARED((N_NODES, F), jnp.float32),
        pltpu.VMEM((IDX_PER_W,), jnp.int32),
        pltpu.VMEM((GROWS, F), jnp.float32),
        pltpu.VMEM((GROWS, F), jnp.float32),
        pltpu.VMEM((GN, F), jnp.float32),
        pltpu.VMEM((GN, F), jnp.float32),
        pltpu.SemaphoreType.DMA,
        pltpu.SemaphoreType.DMA,
        pltpu.SemaphoreType.DMA,
        pltpu.SemaphoreType.DMA,
    ],
)(_sc_body)


def _tc_body(x_ref, nei_ref, ws_ref, wn_ref, b_ref, o_ref):
    acc = lax.dot_general(x_ref[...], ws_ref[...], (((1,), (0,)), ((), ())),
                          precision=lax.Precision.HIGHEST,
                          preferred_element_type=jnp.float32)
    acc = acc + lax.dot_general(nei_ref[...], wn_ref[...],
                                (((1,), (0,)), ((), ())),
                                precision=lax.Precision.HIGHEST,
                                preferred_element_type=jnp.float32)
    o_ref[...] = jnp.maximum(acc + b_ref[...], 0.0)


def _tc_fused(x2, nei_sum, W_self, W_nei_scaled, bias, n_rows, bm):
    grid = (n_rows // bm,)
    return pl.pallas_call(
        _tc_body,
        grid=grid,
        in_specs=[
            pl.BlockSpec((bm, F), lambda i: (i, 0)),
            pl.BlockSpec((bm, F), lambda i: (i, 0)),
            pl.BlockSpec((F, F), lambda i: (0, 0)),
            pl.BlockSpec((F, F), lambda i: (0, 0)),
            pl.BlockSpec((1, F), lambda i: (0, 0)),
        ],
        out_specs=pl.BlockSpec((bm, F), lambda i: (i, 0)),
        out_shape=jax.ShapeDtypeStruct((n_rows, F), jnp.float32),
    )(x2, nei_sum, W_self, W_nei_scaled, bias)


def kernel(x, adj, W_self, b_self, W_nei, b_nei):
    B, N, Fd = x.shape
    Kd = adj.shape[-1]
    x2 = x.reshape(N, Fd)
    adj_flat = adj.astype(jnp.int32).reshape(N * Kd)
    nei_sum = _sc_gather_sum(x2, adj_flat)
    out = _tc_fused(x2, nei_sum, W_self, W_nei * (1.0 / Kd),
                    (b_self + b_nei).reshape(1, Fd), N, 2000)
    return out.reshape(B, N, Fd)


# speedup vs baseline: 45.9145x; 1.0237x over previous
import functools

import jax
import jax.numpy as jnp
from jax import lax
from jax.experimental import pallas as pl
from jax.experimental.pallas import tpu as pltpu
from jax.experimental.pallas import tpu_sc as plsc

F = 128
K = 32
N_NODES = 10000
NC = 2
NS = 16
NW = NC * NS
NODES_PER_W = 320
GROWS = 64
GN = GROWS // K
NSTEPS = NODES_PER_W // GN
RING = 4
NOUTER = NSTEPS // RING
NOUTER_LAST = (N_NODES - (NW - 1) * NODES_PER_W) // GN // RING
KU = 4
LANES = 16
VPR = F // LANES
ROWS_PER_TILE = 640
LAST_TILE_ROWS = N_NODES - 15 * ROWS_PER_TILE
IDX_PER_W = NODES_PER_W * K
IDX_PER_W_LAST = (N_NODES - (NW - 1) * NODES_PER_W) * K


def _sc_body(x_hbm, adj_hbm, out_hbm, x_sh, idx_all,
             rows0, rows1, rows2, rows3, ob0, ob1, ob2, ob3,
             sem0, sem1, sem2, sem3, osem0, osem1, osem2, osem3):
    rows = (rows0, rows1, rows2, rows3)
    sems = (sem0, sem1, sem2, sem3)
    obs = (ob0, ob1, ob2, ob3)
    osems = (osem0, osem1, osem2, osem3)
    sid = lax.axis_index("s")
    wid = sid * NC + lax.axis_index("c")
    node_base = wid * NODES_PER_W
    n_outer = jnp.where(wid == NW - 1, NOUTER_LAST, NOUTER)

    @pl.when(sid < NS - 1)
    def _():
        pltpu.sync_copy(x_hbm.at[pl.ds(sid * ROWS_PER_TILE, ROWS_PER_TILE)],
                        x_sh.at[pl.ds(sid * ROWS_PER_TILE, ROWS_PER_TILE)])

    @pl.when(sid == NS - 1)
    def _():
        pltpu.sync_copy(
            x_hbm.at[pl.ds((NS - 1) * ROWS_PER_TILE, LAST_TILE_ROWS)],
            x_sh.at[pl.ds((NS - 1) * ROWS_PER_TILE, LAST_TILE_ROWS)])

    @pl.when(wid < NW - 1)
    def _():
        pltpu.sync_copy(adj_hbm.at[pl.ds(wid * IDX_PER_W, IDX_PER_W)],
                        idx_all)

    @pl.when(wid == NW - 1)
    def _():
        pltpu.sync_copy(
            adj_hbm.at[pl.ds((NW - 1) * IDX_PER_W, IDX_PER_W_LAST)],
            idx_all.at[pl.ds(0, IDX_PER_W_LAST)])
    plsc.subcore_barrier()

    def start(g, b):
        pltpu.async_copy(x_sh.at[idx_all.at[pl.ds(g * GROWS, GROWS)]],
                         rows[b], sems[b])

    for b in range(RING):
        start(b, b)

    def outer(gb, carry):
        for b in range(RING):
            g = gb * RING + b
            pltpu.make_async_copy(
                x_hbm.at[pl.ds(0, GROWS)], rows[b], sems[b]).wait()

            @pl.when(gb > 0)
            def _(b=b):
                pltpu.make_async_copy(
                    obs[b], out_hbm.at[pl.ds(node_base, GN)],
                    osems[b]).wait()

            for n in range(GN):
                def k_fn(kq, accs, b=b, n=n):
                    accs = list(accs)
                    for u in range(KU):
                        r = n * K + kq * KU + u
                        for j in range(VPR):
                            accs[j] = accs[j] + rows[b][
                                r, pl.ds(j * LANES, LANES)]
                    return tuple(accs)
                accs = lax.fori_loop(
                    0, K // KU, k_fn,
                    tuple(jnp.zeros((LANES,), jnp.float32)
                          for _ in range(VPR)))
                for j in range(VPR):
                    obs[b][n, pl.ds(j * LANES, LANES)] = accs[j]
            pltpu.async_copy(
                obs[b], out_hbm.at[pl.ds(node_base + g * GN, GN)], osems[b])

            @pl.when(gb < n_outer - 1)
            def _(g=g, b=b):
                start(g + RING, b)
        return carry

    lax.fori_loop(0, n_outer, outer, 0)
    for b in range(RING):
        pltpu.make_async_copy(
            obs[b], out_hbm.at[pl.ds(node_base, GN)], osems[b]).wait()


_sc_gather_sum = functools.partial(
    pl.kernel,
    mesh=plsc.VectorSubcoreMesh(core_axis_name="c", subcore_axis_name="s"),
    out_type=jax.ShapeDtypeStruct((N_NODES, F), jnp.float32),
    scratch_types=[
        pltpu.VMEM_SHARED((N_NODES, F), jnp.float32),
        pltpu.VMEM((IDX_PER_W,), jnp.int32),
        pltpu.VMEM((GROWS, F), jnp.float32),
        pltpu.VMEM((GROWS, F), jnp.float32),
        pltpu.VMEM((GROWS, F), jnp.float32),
        pltpu.VMEM((GROWS, F), jnp.float32),
        pltpu.VMEM((GN, F), jnp.float32),
        pltpu.VMEM((GN, F), jnp.float32),
        pltpu.VMEM((GN, F), jnp.float32),
        pltpu.VMEM((GN, F), jnp.float32),
        pltpu.SemaphoreType.DMA,
        pltpu.SemaphoreType.DMA,
        pltpu.SemaphoreType.DMA,
        pltpu.SemaphoreType.DMA,
        pltpu.SemaphoreType.DMA,
        pltpu.SemaphoreType.DMA,
        pltpu.SemaphoreType.DMA,
        pltpu.SemaphoreType.DMA,
    ],
)(_sc_body)


def _tc_body(x_ref, nei_ref, ws_ref, wn_ref, b_ref, o_ref):
    acc = lax.dot_general(x_ref[...], ws_ref[...], (((1,), (0,)), ((), ())),
                          precision=lax.Precision.HIGHEST,
                          preferred_element_type=jnp.float32)
    acc = acc + lax.dot_general(nei_ref[...], wn_ref[...],
                                (((1,), (0,)), ((), ())),
                                precision=lax.Precision.HIGHEST,
                                preferred_element_type=jnp.float32)
    o_ref[...] = jnp.maximum(acc + b_ref[...], 0.0)


def _tc_fused(x2, nei_sum, W_self, W_nei_scaled, bias, n_rows, bm):
    grid = (n_rows // bm,)
    return pl.pallas_call(
        _tc_body,
        grid=grid,
        in_specs=[
            pl.BlockSpec((bm, F), lambda i: (i, 0)),
            pl.BlockSpec((bm, F), lambda i: (i, 0)),
            pl.BlockSpec((F, F), lambda i: (0, 0)),
            pl.BlockSpec((F, F), lambda i: (0, 0)),
            pl.BlockSpec((1, F), lambda i: (0, 0)),
        ],
        out_specs=pl.BlockSpec((bm, F), lambda i: (i, 0)),
        out_shape=jax.ShapeDtypeStruct((n_rows, F), jnp.float32),
    )(x2, nei_sum, W_self, W_nei_scaled, bias)


def kernel(x, adj, W_self, b_self, W_nei, b_nei):
    B, N, Fd = x.shape
    Kd = adj.shape[-1]
    x2 = x.reshape(N, Fd)
    adj_flat = adj.astype(jnp.int32).reshape(N * Kd)
    nei_sum = _sc_gather_sum(x2, adj_flat)
    out = _tc_fused(x2, nei_sum, W_self, W_nei * (1.0 / Kd),
                    (b_self + b_nei).reshape(1, Fd), N, 2000)
    return out.reshape(B, N, Fd)


# speedup vs baseline: 47.0896x; 1.0256x over previous
import functools

import jax
import jax.numpy as jnp
from jax import lax
from jax.experimental import pallas as pl
from jax.experimental.pallas import tpu as pltpu
from jax.experimental.pallas import tpu_sc as plsc

F = 128
K = 32
N_NODES = 10000
NC = 2
NS = 16
NW = NC * NS
NODES_PER_W = 320
GROWS = 64
GN = GROWS // K
NSTEPS = NODES_PER_W // GN
RING = 4
NOUTER = NSTEPS // RING
NOUTER_LAST = (N_NODES - (NW - 1) * NODES_PER_W) // GN // RING
KU = 4
LANES = 16
VPR = F // LANES
ROWS_PER_TILE = 640
LAST_TILE_ROWS = N_NODES - 15 * ROWS_PER_TILE
IDX_PER_W = NODES_PER_W * K
IDX_PER_W_LAST = (N_NODES - (NW - 1) * NODES_PER_W) * K


def _sc_body(x_hbm, adj_hbm, out_hbm, x_sh, idx_all,
             rows0, rows1, rows2, rows3, ob0, ob1, ob2, ob3,
             sem0, sem1, sem2, sem3, osem0, osem1, osem2, osem3):
    rows = (rows0, rows1, rows2, rows3)
    sems = (sem0, sem1, sem2, sem3)
    obs = (ob0, ob1, ob2, ob3)
    osems = (osem0, osem1, osem2, osem3)
    sid = lax.axis_index("s")
    wid = sid * NC + lax.axis_index("c")
    node_base = wid * NODES_PER_W
    n_outer = jnp.where(wid == NW - 1, NOUTER_LAST, NOUTER)

    @pl.when(sid < NS - 1)
    def _():
        pltpu.sync_copy(x_hbm.at[pl.ds(sid * ROWS_PER_TILE, ROWS_PER_TILE)],
                        x_sh.at[pl.ds(sid * ROWS_PER_TILE, ROWS_PER_TILE)])

    @pl.when(sid == NS - 1)
    def _():
        pltpu.sync_copy(
            x_hbm.at[pl.ds((NS - 1) * ROWS_PER_TILE, LAST_TILE_ROWS)],
            x_sh.at[pl.ds((NS - 1) * ROWS_PER_TILE, LAST_TILE_ROWS)])

    @pl.when(wid < NW - 1)
    def _():
        pltpu.sync_copy(adj_hbm.at[pl.ds(wid * IDX_PER_W, IDX_PER_W)],
                        idx_all)

    @pl.when(wid == NW - 1)
    def _():
        pltpu.sync_copy(
            adj_hbm.at[pl.ds((NW - 1) * IDX_PER_W, IDX_PER_W_LAST)],
            idx_all.at[pl.ds(0, IDX_PER_W_LAST)])
    plsc.subcore_barrier()

    def start(g, b):
        pltpu.async_copy(x_sh.at[idx_all.at[pl.ds(g * GROWS, GROWS)]],
                         rows[b], sems[b])

    for b in range(RING):
        start(b, b)

    def outer(gb, carry):
        for b in range(RING):
            g = gb * RING + b
            pltpu.make_async_copy(
                x_hbm.at[pl.ds(0, GROWS)], rows[b], sems[b]).wait()

            @pl.when(gb > 0)
            def _(b=b):
                pltpu.make_async_copy(
                    obs[b], out_hbm.at[pl.ds(node_base, GN)],
                    osems[b]).wait()

            for n in range(GN):
                def k_fn(kq, accs, b=b, n=n):
                    accs = list(accs)
                    for u in range(KU):
                        r = n * K + kq * KU + u
                        for j in range(VPR):
                            accs[j] = accs[j] + rows[b][
                                r, pl.ds(j * LANES, LANES)]
                    return tuple(accs)
                accs = lax.fori_loop(
                    0, K // KU, k_fn,
                    tuple(jnp.zeros((LANES,), jnp.float32)
                          for _ in range(VPR)))
                for j in range(VPR):
                    obs[b][n, pl.ds(j * LANES, LANES)] = accs[j]
            pltpu.async_copy(
                obs[b], out_hbm.at[pl.ds(node_base + g * GN, GN)], osems[b])

            @pl.when(gb < n_outer - 1)
            def _(g=g, b=b):
                start(g + RING, b)
        return carry

    lax.fori_loop(0, n_outer, outer, 0)
    for b in range(RING):
        pltpu.make_async_copy(
            obs[b], out_hbm.at[pl.ds(node_base, GN)], osems[b]).wait()


_sc_gather_sum = functools.partial(
    pl.kernel,
    mesh=plsc.VectorSubcoreMesh(core_axis_name="c", subcore_axis_name="s"),
    out_type=jax.ShapeDtypeStruct((N_NODES, F), jnp.float32),
    scratch_types=[
        pltpu.VMEM_SHARED((N_NODES, F), jnp.float32),
        pltpu.VMEM((IDX_PER_W,), jnp.int32),
        pltpu.VMEM((GROWS, F), jnp.float32),
        pltpu.VMEM((GROWS, F), jnp.float32),
        pltpu.VMEM((GROWS, F), jnp.float32),
        pltpu.VMEM((GROWS, F), jnp.float32),
        pltpu.VMEM((GN, F), jnp.float32),
        pltpu.VMEM((GN, F), jnp.float32),
        pltpu.VMEM((GN, F), jnp.float32),
        pltpu.VMEM((GN, F), jnp.float32),
        pltpu.SemaphoreType.DMA,
        pltpu.SemaphoreType.DMA,
        pltpu.SemaphoreType.DMA,
        pltpu.SemaphoreType.DMA,
        pltpu.SemaphoreType.DMA,
        pltpu.SemaphoreType.DMA,
        pltpu.SemaphoreType.DMA,
        pltpu.SemaphoreType.DMA,
    ],
)(_sc_body)


def _tc_self_body(x_ref, ws_ref, b_ref, o_ref):
    o_ref[...] = lax.dot_general(
        x_ref[...], ws_ref[...], (((1,), (0,)), ((), ())),
        precision=lax.Precision.HIGHEST,
        preferred_element_type=jnp.float32) + b_ref[...]


def _tc_self(x2, W_self, bias, n_rows, bm):
    return pl.pallas_call(
        _tc_self_body,
        grid=(n_rows // bm,),
        in_specs=[
            pl.BlockSpec((bm, F), lambda i: (i, 0)),
            pl.BlockSpec((F, F), lambda i: (0, 0)),
            pl.BlockSpec((1, F), lambda i: (0, 0)),
        ],
        out_specs=pl.BlockSpec((bm, F), lambda i: (i, 0)),
        out_shape=jax.ShapeDtypeStruct((n_rows, F), jnp.float32),
    )(x2, W_self, bias)


def _tc_comb_body(self_ref, nei_ref, wn_ref, o_ref):
    acc = self_ref[...] + lax.dot_general(
        nei_ref[...], wn_ref[...], (((1,), (0,)), ((), ())),
        precision=lax.Precision.HIGHEST,
        preferred_element_type=jnp.float32)
    o_ref[...] = jnp.maximum(acc, 0.0)


def _tc_comb(selfp, nei_sum, W_nei_scaled, n_rows, bm):
    return pl.pallas_call(
        _tc_comb_body,
        grid=(n_rows // bm,),
        in_specs=[
            pl.BlockSpec((bm, F), lambda i: (i, 0)),
            pl.BlockSpec((bm, F), lambda i: (i, 0)),
            pl.BlockSpec((F, F), lambda i: (0, 0)),
        ],
        out_specs=pl.BlockSpec((bm, F), lambda i: (i, 0)),
        out_shape=jax.ShapeDtypeStruct((n_rows, F), jnp.float32),
    )(selfp, nei_sum, W_nei_scaled)


def kernel(x, adj, W_self, b_self, W_nei, b_nei):
    B, N, Fd = x.shape
    Kd = adj.shape[-1]
    x2 = x.reshape(N, Fd)
    adj_flat = adj.astype(jnp.int32).reshape(N * Kd)
    nei_sum = _sc_gather_sum(x2, adj_flat)
    selfp = _tc_self(x2, W_self, (b_self + b_nei).reshape(1, Fd), N, 2000)
    out = _tc_comb(selfp, nei_sum, W_nei * (1.0 / Kd), N, 2000)
    return out.reshape(B, N, Fd)


# speedup vs baseline: 47.9804x; 1.0189x over previous
import functools

import jax
import jax.numpy as jnp
from jax import lax
from jax.experimental import pallas as pl
from jax.experimental.pallas import tpu as pltpu
from jax.experimental.pallas import tpu_sc as plsc

F = 128
K = 32
N_NODES = 10000
NC = 2
NS = 16
NW = NC * NS
NODES_PER_W = 320
GROWS = 64
GN = GROWS // K
NSTEPS = NODES_PER_W // GN
RING = 4
NOUTER = NSTEPS // RING
NOUTER_LAST = (N_NODES - (NW - 1) * NODES_PER_W) // GN // RING
KU = 4
LANES = 16
VPR = F // LANES
ROWS_PER_TILE = 640
LAST_TILE_ROWS = N_NODES - 15 * ROWS_PER_TILE
IDX_PER_W = NODES_PER_W * K
IDX_PER_W_LAST = (N_NODES - (NW - 1) * NODES_PER_W) * K


def _sc_body(x_hbm, adj_hbm, out_hbm, x_sh, idx_all,
             rows0, rows1, rows2, rows3, ob0, ob1, ob2, ob3,
             sem0, sem1, sem2, sem3, osem0, osem1, osem2, osem3):
    rows = (rows0, rows1, rows2, rows3)
    sems = (sem0, sem1, sem2, sem3)
    obs = (ob0, ob1, ob2, ob3)
    osems = (osem0, osem1, osem2, osem3)
    sid = lax.axis_index("s")
    wid = sid * NC + lax.axis_index("c")
    node_base = wid * NODES_PER_W
    n_outer = jnp.where(wid == NW - 1, NOUTER_LAST, NOUTER)

    @pl.when(sid < NS - 1)
    def _():
        pltpu.sync_copy(x_hbm.at[pl.ds(sid * ROWS_PER_TILE, ROWS_PER_TILE)],
                        x_sh.at[pl.ds(sid * ROWS_PER_TILE, ROWS_PER_TILE)])

    @pl.when(sid == NS - 1)
    def _():
        pltpu.sync_copy(
            x_hbm.at[pl.ds((NS - 1) * ROWS_PER_TILE, LAST_TILE_ROWS)],
            x_sh.at[pl.ds((NS - 1) * ROWS_PER_TILE, LAST_TILE_ROWS)])

    @pl.when(wid < NW - 1)
    def _():
        pltpu.sync_copy(adj_hbm.at[pl.ds(wid * IDX_PER_W, IDX_PER_W)],
                        idx_all)

    @pl.when(wid == NW - 1)
    def _():
        pltpu.sync_copy(
            adj_hbm.at[pl.ds((NW - 1) * IDX_PER_W, IDX_PER_W_LAST)],
            idx_all.at[pl.ds(0, IDX_PER_W_LAST)])
    plsc.subcore_barrier()

    def start(g, b):
        pltpu.async_copy(x_sh.at[idx_all.at[pl.ds(g * GROWS, GROWS)]],
                         rows[b], sems[b])

    for b in range(RING):
        start(b, b)

    def outer(gb, carry):
        for b in range(RING):
            g = gb * RING + b
            pltpu.make_async_copy(
                x_hbm.at[pl.ds(0, GROWS)], rows[b], sems[b]).wait()

            @pl.when(gb > 0)
            def _(b=b):
                pltpu.make_async_copy(
                    obs[b], out_hbm.at[pl.ds(node_base, GN)],
                    osems[b]).wait()

            for n in range(GN):
                def k_fn(kq, accs, b=b, n=n):
                    accs = list(accs)
                    for u in range(KU):
                        r = n * K + kq * KU + u
                        for j in range(VPR):
                            accs[j] = accs[j] + rows[b][
                                r, pl.ds(j * LANES, LANES)]
                    return tuple(accs)
                accs = lax.fori_loop(
                    0, K // KU, k_fn,
                    tuple(jnp.zeros((LANES,), jnp.float32)
                          for _ in range(VPR)))
                for j in range(VPR):
                    obs[b][n, pl.ds(j * LANES, LANES)] = accs[j]
            pltpu.async_copy(
                obs[b], out_hbm.at[pl.ds(node_base + g * GN, GN)], osems[b])

            @pl.when(gb < n_outer - 1)
            def _(g=g, b=b):
                start(g + RING, b)
        return carry

    lax.fori_loop(0, n_outer, outer, 0)
    for b in range(RING):
        pltpu.make_async_copy(
            obs[b], out_hbm.at[pl.ds(node_base, GN)], osems[b]).wait()


_sc_gather_sum = functools.partial(
    pl.kernel,
    mesh=plsc.VectorSubcoreMesh(core_axis_name="c", subcore_axis_name="s"),
    out_type=jax.ShapeDtypeStruct((N_NODES, F), jnp.float32),
    scratch_types=[
        pltpu.VMEM_SHARED((N_NODES, F), jnp.float32),
        pltpu.VMEM((IDX_PER_W,), jnp.int32),
        pltpu.VMEM((GROWS, F), jnp.float32),
        pltpu.VMEM((GROWS, F), jnp.float32),
        pltpu.VMEM((GROWS, F), jnp.float32),
        pltpu.VMEM((GROWS, F), jnp.float32),
        pltpu.VMEM((GN, F), jnp.float32),
        pltpu.VMEM((GN, F), jnp.float32),
        pltpu.VMEM((GN, F), jnp.float32),
        pltpu.VMEM((GN, F), jnp.float32),
        pltpu.SemaphoreType.DMA,
        pltpu.SemaphoreType.DMA,
        pltpu.SemaphoreType.DMA,
        pltpu.SemaphoreType.DMA,
        pltpu.SemaphoreType.DMA,
        pltpu.SemaphoreType.DMA,
        pltpu.SemaphoreType.DMA,
        pltpu.SemaphoreType.DMA,
    ],
)(_sc_body)


def _tc_self_body(x_ref, ws_ref, b_ref, o_ref):
    o_ref[...] = lax.dot_general(
        x_ref[...], ws_ref[...], (((1,), (0,)), ((), ())),
        precision=lax.Precision.HIGHEST,
        preferred_element_type=jnp.float32) + b_ref[...]


def _tc_self(x2, W_self, bias, n_rows, bm):
    return pl.pallas_call(
        _tc_self_body,
        grid=(n_rows // bm,),
        in_specs=[
            pl.BlockSpec((bm, F), lambda i: (i, 0)),
            pl.BlockSpec((F, F), lambda i: (0, 0)),
            pl.BlockSpec((1, F), lambda i: (0, 0)),
        ],
        out_specs=pl.BlockSpec((bm, F), lambda i: (i, 0)),
        out_shape=jax.ShapeDtypeStruct((n_rows, F), jnp.float32),
    )(x2, W_self, bias)


def _tc_comb_body(self_ref, nei_ref, wn_ref, o_ref):
    acc = self_ref[...] + lax.dot_general(
        nei_ref[...], wn_ref[...], (((1,), (0,)), ((), ())),
        precision=lax.Precision.DEFAULT,
        preferred_element_type=jnp.float32)
    o_ref[...] = jnp.maximum(acc, 0.0)


def _tc_comb(selfp, nei_sum, W_nei_scaled, n_rows, bm):
    return pl.pallas_call(
        _tc_comb_body,
        grid=(n_rows // bm,),
        in_specs=[
            pl.BlockSpec((bm, F), lambda i: (i, 0)),
            pl.BlockSpec((bm, F), lambda i: (i, 0)),
            pl.BlockSpec((F, F), lambda i: (0, 0)),
        ],
        out_specs=pl.BlockSpec((bm, F), lambda i: (i, 0)),
        out_shape=jax.ShapeDtypeStruct((n_rows, F), jnp.float32),
    )(selfp, nei_sum, W_nei_scaled)


def kernel(x, adj, W_self, b_self, W_nei, b_nei):
    B, N, Fd = x.shape
    Kd = adj.shape[-1]
    x2 = x.reshape(N, Fd)
    adj_flat = adj.astype(jnp.int32).reshape(N * Kd)
    nei_sum = _sc_gather_sum(x2, adj_flat)
    selfp = _tc_self(x2, W_self, (b_self + b_nei).reshape(1, Fd), N, 2000)
    out = _tc_comb(selfp, nei_sum, W_nei * (1.0 / Kd), N, 2000)
    return out.reshape(B, N, Fd)
